# Initial kernel scaffold; baseline (speedup 1.0000x reference)
import jax
import jax.numpy as jnp
from jax.experimental import pallas as pl


def kernel(x1, x2, cheb_W0, cheb_W1, cheb_b, W_ih_l0, W_hh_l0, b_ih_l0, b_hh_l0, W_ih_l1, W_hh_l1, b_ih_l1, b_hh_l1, fc_W, fc_b):
    raise NotImplementedError("write your pallas kernel here")


# speedup vs baseline: 2.6477x; 2.6477x over previous
import numpy as np
import jax
import jax.numpy as jnp
from jax.experimental import pallas as pl
from jax.experimental.pallas import tpu as pltpu

_EDGE_SRC = np.array([0, 2, 4, 1, 3, 5, 6, 8, 10, 7, 9, 11, 13, 12, 2, 4, 14, 3, 5, 14, 8, 10, 13, 9, 11, 13, 12, 13], dtype=np.int64)
_EDGE_DST = np.array([2, 4, 14, 3, 5, 14, 8, 10, 13, 9, 11, 13, 14, 13, 0, 2, 4, 1, 3, 5, 6, 8, 10, 7, 9, 11, 13, 14], dtype=np.int64)
_V = 15


def _lhat() -> np.ndarray:
    deg = np.zeros((_V,), np.float64)
    for s in _EDGE_SRC:
        deg[s] += 1.0
    dinv = np.where(deg > 0, 1.0 / np.sqrt(np.maximum(deg, 1e-12)), 0.0)
    L = np.zeros((_V, _V), np.float64)
    for s, d in zip(_EDGE_SRC, _EDGE_DST):
        L[d, s] += -dinv[s] * dinv[d]
    return L.astype(np.float32)


_LHAT = _lhat()

_T, _B, _H, _C = 300, 64, 128, 3
_G4 = 4 * _H
_VC = _V * _C
_CH = 60
_NCH = _T // _CH


def _gate(g, c_prev):
    i = jax.nn.sigmoid(g[:, :_H])
    f = jax.nn.sigmoid(g[:, _H:2 * _H])
    gg = jnp.tanh(g[:, 2 * _H:3 * _H])
    o = jax.nn.sigmoid(g[:, 3 * _H:])
    c = f * c_prev + i * gg
    h = o * jnp.tanh(c)
    return h, c


def _lstm_kernel(xin_ref, p0_ref, c0_ref, wf_ref, c1_ref, fcw_ref, fcb_ref,
                 out_ref, gc_ref):
    p0 = p0_ref[...]
    c0b = c0_ref[...]
    wf = wf_ref[...]
    c1b = c1_ref[...]
    z = jnp.zeros((_B, _H), jnp.float32)

    def chunk(ci, carry):
        base = ci * (_CH * _B)
        gc_ref[...] = jnp.dot(xin_ref[pl.ds(base, _CH * _B), :], p0,
                              preferred_element_type=jnp.float32) + c0b

        def step(j, st):
            h0, c0s, h1, c1s = st
            hcat = jnp.concatenate([h0, h1], axis=1)
            gall = jnp.dot(hcat, wf, preferred_element_type=jnp.float32)
            g0 = gc_ref[pl.ds(j * _B, _B), :] + gall[:, :_G4]
            g1 = gall[:, _G4:] + c1b
            h0n, c0n = _gate(g0, c0s)
            h1n, c1n = _gate(g1, c1s)
            first = (ci * _CH + j) == 0
            h1 = jnp.where(first, h1, h1n)
            c1s = jnp.where(first, c1s, c1n)
            return (h0n, c0n, h1, c1s)

        return jax.lax.fori_loop(0, _CH, step, carry)

    h0, c0s, h1, c1s = jax.lax.fori_loop(0, _NCH, chunk, (z, z, z, z))

    hcat = jnp.concatenate([h0, h1], axis=1)
    gall = jnp.dot(hcat, wf, preferred_element_type=jnp.float32)
    g1 = gall[:, _G4:] + c1b
    h1, _ = _gate(g1, c1s)

    logits = jnp.dot(h1, fcw_ref[...],
                     preferred_element_type=jnp.float32) + fcb_ref[...]
    m = jnp.max(logits, axis=1, keepdims=True)
    e = jnp.exp(logits - m)
    out_ref[...] = e / jnp.sum(e, axis=1, keepdims=True)


def kernel(x1, x2, cheb_W0, cheb_W1, cheb_b, W_ih_l0, W_hh_l0, b_ih_l0,
           b_hh_l0, W_ih_l1, W_hh_l1, b_ih_l1, b_hh_l1, fc_W, fc_b):
    del x2
    xin = jnp.transpose(x1, (2, 0, 4, 3, 1)).reshape(_T * _B, _VC)

    lhat = jnp.asarray(_LHAT)
    eye = jnp.eye(_V, dtype=jnp.float32)
    mflat = jnp.kron(eye, cheb_W0.T) + jnp.kron(lhat.T, cheb_W1.T)
    p0 = mflat @ W_ih_l0.T
    c0 = (jnp.tile(cheb_b, _V) @ W_ih_l0.T + b_ih_l0 + b_hh_l0)[None, :]

    wf = jnp.concatenate([
        jnp.concatenate([W_hh_l0.T, W_ih_l1.T], axis=1),
        jnp.concatenate([jnp.zeros((_H, _G4), jnp.float32), W_hh_l1.T], axis=1),
    ], axis=0)
    c1 = (b_ih_l1 + b_hh_l1)[None, :]

    return pl.pallas_call(
        _lstm_kernel,
        out_shape=jax.ShapeDtypeStruct((_B, fc_W.shape[0]), jnp.float32),
        scratch_shapes=[pltpu.VMEM((_CH * _B, _G4), jnp.float32)],
    )(xin, p0, c0, wf, c1, fc_W.T, fc_b[None, :])


# speedup vs baseline: 2.7593x; 1.0422x over previous
import numpy as np
import jax
import jax.numpy as jnp
from jax.experimental import pallas as pl
from jax.experimental.pallas import tpu as pltpu

_EDGE_SRC = np.array([0, 2, 4, 1, 3, 5, 6, 8, 10, 7, 9, 11, 13, 12, 2, 4, 14, 3, 5, 14, 8, 10, 13, 9, 11, 13, 12, 13], dtype=np.int64)
_EDGE_DST = np.array([2, 4, 14, 3, 5, 14, 8, 10, 13, 9, 11, 13, 14, 13, 0, 2, 4, 1, 3, 5, 6, 8, 10, 7, 9, 11, 13, 14], dtype=np.int64)
_V = 15


def _lhat() -> np.ndarray:
    deg = np.zeros((_V,), np.float64)
    for s in _EDGE_SRC:
        deg[s] += 1.0
    dinv = np.where(deg > 0, 1.0 / np.sqrt(np.maximum(deg, 1e-12)), 0.0)
    L = np.zeros((_V, _V), np.float64)
    for s, d in zip(_EDGE_SRC, _EDGE_DST):
        L[d, s] += -dinv[s] * dinv[d]
    return L.astype(np.float32)


_LHAT = _lhat()

_T, _B, _H, _C = 300, 64, 128, 3
_G4 = 4 * _H
_VC = _V * _C
_CH = 60
_NCH = _T // _CH


def _gate(g, c_prev):
    i = jax.nn.sigmoid(g[:, :_H])
    f = jax.nn.sigmoid(g[:, _H:2 * _H])
    gg = jnp.tanh(g[:, 2 * _H:3 * _H])
    o = jax.nn.sigmoid(g[:, 3 * _H:])
    c = f * c_prev + i * gg
    h = o * jnp.tanh(c)
    return h, c


def _lstm_kernel(xin_ref, p0_ref, c0_ref, wf_ref, c1_ref, fcw_ref, fcb_ref,
                 out_ref, gc_ref):
    p0 = p0_ref[...]
    c0b = c0_ref[...]
    wf = wf_ref[...]
    c1b = c1_ref[...]
    z = jnp.zeros((_B, _H), jnp.float32)

    def chunk(ci, carry):
        base = ci * (_CH * _B)
        gc_ref[...] = jnp.dot(xin_ref[pl.ds(base, _CH * _B), :], p0,
                              preferred_element_type=jnp.float32) + c0b

        def step(j, st):
            h0, c0s, h1, c1s = st
            hcat = jnp.concatenate([h0, h1], axis=1).astype(jnp.bfloat16)
            gall = jnp.dot(hcat, wf, preferred_element_type=jnp.float32)
            g0 = gc_ref[pl.ds(j * _B, _B), :] + gall[:, :_G4]
            g1 = gall[:, _G4:] + c1b
            h0n, c0n = _gate(g0, c0s)
            h1n, c1n = _gate(g1, c1s)
            first = (ci * _CH + j) == 0
            h1 = jnp.where(first, h1, h1n)
            c1s = jnp.where(first, c1s, c1n)
            return (h0n, c0n, h1, c1s)

        return jax.lax.fori_loop(0, _CH, step, carry)

    h0, c0s, h1, c1s = jax.lax.fori_loop(0, _NCH, chunk, (z, z, z, z))

    hcat = jnp.concatenate([h0, h1], axis=1).astype(jnp.bfloat16)
    gall = jnp.dot(hcat, wf, preferred_element_type=jnp.float32)
    g1 = gall[:, _G4:] + c1b
    h1, _ = _gate(g1, c1s)

    logits = jnp.dot(h1, fcw_ref[...],
                     preferred_element_type=jnp.float32) + fcb_ref[...]
    m = jnp.max(logits, axis=1, keepdims=True)
    e = jnp.exp(logits - m)
    out_ref[...] = e / jnp.sum(e, axis=1, keepdims=True)


def kernel(x1, x2, cheb_W0, cheb_W1, cheb_b, W_ih_l0, W_hh_l0, b_ih_l0,
           b_hh_l0, W_ih_l1, W_hh_l1, b_ih_l1, b_hh_l1, fc_W, fc_b):
    del x2
    xin = jnp.transpose(x1, (2, 0, 4, 3, 1)).reshape(_T * _B, _VC)
    xin = xin.astype(jnp.bfloat16)

    lhat = jnp.asarray(_LHAT)
    eye = jnp.eye(_V, dtype=jnp.float32)
    mflat = jnp.kron(eye, cheb_W0.T) + jnp.kron(lhat.T, cheb_W1.T)
    p0 = (mflat @ W_ih_l0.T).astype(jnp.bfloat16)
    c0 = (jnp.tile(cheb_b, _V) @ W_ih_l0.T + b_ih_l0 + b_hh_l0)[None, :]

    wf = jnp.concatenate([
        jnp.concatenate([W_hh_l0.T, W_ih_l1.T], axis=1),
        jnp.concatenate([jnp.zeros((_H, _G4), jnp.float32), W_hh_l1.T], axis=1),
    ], axis=0).astype(jnp.bfloat16)
    c1 = (b_ih_l1 + b_hh_l1)[None, :]

    return pl.pallas_call(
        _lstm_kernel,
        out_shape=jax.ShapeDtypeStruct((_B, fc_W.shape[0]), jnp.float32),
        scratch_shapes=[pltpu.VMEM((_CH * _B, _G4), jnp.float32)],
    )(xin, p0, c0, wf, c1, fc_W.T, fc_b[None, :])


# speedup vs baseline: 2.8371x; 1.0282x over previous
import numpy as np
import jax
import jax.numpy as jnp
from jax.experimental import pallas as pl
from jax.experimental.pallas import tpu as pltpu

_EDGE_SRC = np.array([0, 2, 4, 1, 3, 5, 6, 8, 10, 7, 9, 11, 13, 12, 2, 4, 14, 3, 5, 14, 8, 10, 13, 9, 11, 13, 12, 13], dtype=np.int64)
_EDGE_DST = np.array([2, 4, 14, 3, 5, 14, 8, 10, 13, 9, 11, 13, 14, 13, 0, 2, 4, 1, 3, 5, 6, 8, 10, 7, 9, 11, 13, 14], dtype=np.int64)
_V = 15


def _lhat() -> np.ndarray:
    deg = np.zeros((_V,), np.float64)
    for s in _EDGE_SRC:
        deg[s] += 1.0
    dinv = np.where(deg > 0, 1.0 / np.sqrt(np.maximum(deg, 1e-12)), 0.0)
    L = np.zeros((_V, _V), np.float64)
    for s, d in zip(_EDGE_SRC, _EDGE_DST):
        L[d, s] += -dinv[s] * dinv[d]
    return L.astype(np.float32)


_LHAT = _lhat()

_T, _B, _H, _C = 300, 64, 128, 3
_G4 = 4 * _H
_VC = _V * _C
_CH = 60
_NCH = _T // _CH


def _gate(g, c_prev):
    i = jax.nn.sigmoid(g[:, :_H])
    f = jax.nn.sigmoid(g[:, _H:2 * _H])
    gg = jnp.tanh(g[:, 2 * _H:3 * _H])
    o = jax.nn.sigmoid(g[:, 3 * _H:])
    c = f * c_prev + i * gg
    h = o * jnp.tanh(c)
    return h, c


def _lstm_kernel(xin_ref, p0_ref, c0_ref, wf_ref, c1_ref, fcw_ref, fcb_ref,
                 out_ref, gc_ref):
    z = jnp.zeros((_B, _H), jnp.float32)

    def substep(i_glob, j, st):
        h0, c0s, h1, c1s = st
        hcat = jnp.concatenate([h0, h1], axis=1).astype(jnp.bfloat16)
        gall = jnp.dot(hcat, wf_ref[...], preferred_element_type=jnp.float32)
        g0 = gc_ref[pl.ds(j * _B, _B), :] + gall[:, :_G4]
        g1 = gall[:, _G4:] + c1_ref[...]
        h0n, c0n = _gate(g0, c0s)
        h1n, c1n = _gate(g1, c1s)
        first = i_glob == 0
        h1 = jnp.where(first, h1, h1n)
        c1s = jnp.where(first, c1s, c1n)
        return (h0n, c0n, h1, c1s)

    def chunk(ci, carry):
        base = ci * (_CH * _B)
        gc_ref[...] = jnp.dot(xin_ref[pl.ds(base, _CH * _B), :], p0_ref[...],
                              preferred_element_type=jnp.float32) + c0_ref[...]

        def step(jj, st):
            st = substep(ci * _CH + 2 * jj, 2 * jj, st)
            return substep(ci * _CH + 2 * jj + 1, 2 * jj + 1, st)

        return jax.lax.fori_loop(0, _CH // 2, step, carry)

    h0, c0s, h1, c1s = jax.lax.fori_loop(0, _NCH, chunk, (z, z, z, z))

    hcat = jnp.concatenate([h0, h1], axis=1).astype(jnp.bfloat16)
    gall = jnp.dot(hcat, wf_ref[...], preferred_element_type=jnp.float32)
    g1 = gall[:, _G4:] + c1_ref[...]
    h1, _ = _gate(g1, c1s)

    logits = jnp.dot(h1, fcw_ref[...],
                     preferred_element_type=jnp.float32) + fcb_ref[...]
    m = jnp.max(logits, axis=1, keepdims=True)
    e = jnp.exp(logits - m)
    out_ref[...] = e / jnp.sum(e, axis=1, keepdims=True)


def kernel(x1, x2, cheb_W0, cheb_W1, cheb_b, W_ih_l0, W_hh_l0, b_ih_l0,
           b_hh_l0, W_ih_l1, W_hh_l1, b_ih_l1, b_hh_l1, fc_W, fc_b):
    del x2
    xin = jnp.transpose(x1, (2, 0, 4, 3, 1)).reshape(_T * _B, _VC)
    xin = xin.astype(jnp.bfloat16)

    lhat = jnp.asarray(_LHAT)
    eye = jnp.eye(_V, dtype=jnp.float32)
    mflat = jnp.kron(eye, cheb_W0.T) + jnp.kron(lhat.T, cheb_W1.T)
    p0 = (mflat @ W_ih_l0.T).astype(jnp.bfloat16)
    c0 = (jnp.tile(cheb_b, _V) @ W_ih_l0.T + b_ih_l0 + b_hh_l0)[None, :]

    wf = jnp.concatenate([
        jnp.concatenate([W_hh_l0.T, W_ih_l1.T], axis=1),
        jnp.concatenate([jnp.zeros((_H, _G4), jnp.float32), W_hh_l1.T], axis=1),
    ], axis=0).astype(jnp.bfloat16)
    c1 = (b_ih_l1 + b_hh_l1)[None, :]

    return pl.pallas_call(
        _lstm_kernel,
        out_shape=jax.ShapeDtypeStruct((_B, fc_W.shape[0]), jnp.float32),
        scratch_shapes=[pltpu.VMEM((_CH * _B, _G4), jnp.float32)],
    )(xin, p0, c0, wf, c1, fc_W.T, fc_b[None, :])
